# baseline (speedup 1.0000x reference)
import dataclasses
import functools

import jax
import jax.numpy as jnp
from jax import lax
from jax.experimental import pallas as pl
from jax.experimental.pallas import tpu as pltpu
from jax.experimental.pallas import tpu_sc as plsc

_K = 15
_N = 4096
_C = 64
_B = 8
_R = 512
_RB = 8
_G = 8
_NG = _N // (16 * _G)


def _dist_kernel(xb_ref, xa_ref, d_ref):
    xb = xb_ref[...]
    xa = xa_ref[...]
    sq_all = jnp.sum(xa * xa, axis=0)
    sq_rows = jnp.sum(xb * xb, axis=0)
    g = lax.dot_general(
        xb, xa, (((0,), (0,)), ((), ())),
        preferred_element_type=jnp.float32,
        precision=lax.Precision.DEFAULT,
    )
    d = (sq_rows[:, None] + sq_all[None, :]) - 2.0 * g
    base = pl.program_id(0) * _R
    col = lax.broadcasted_iota(jnp.int32, (_R, _N), 1)
    row_g = lax.broadcasted_iota(jnp.int32, (_R, _N), 0) + base
    d_ref[...] = jnp.where(col == row_g, jnp.inf, d)


def _tc_distance(xb):
    return pl.pallas_call(
        _dist_kernel,
        grid=(_N // _R,),
        in_specs=[
            pl.BlockSpec((_C, _R), lambda r: (0, r)),
            pl.BlockSpec((_C, _N), lambda r: (0, 0)),
        ],
        out_specs=pl.BlockSpec((_R, _N), lambda r: (r, 0)),
        out_shape=jax.ShapeDtypeStruct((_N, _N), jnp.float32),
    )(xb, xb)


def _sc_topk(d):
    mesh = plsc.VectorSubcoreMesh(core_axis_name="c", subcore_axis_name="s")
    cp = pltpu.CompilerParams()
    if "needs_layout_passes" in pltpu.CompilerParams.__dataclass_fields__:
        cp = dataclasses.replace(cp, needs_layout_passes=False)

    @functools.partial(
        pl.kernel,
        out_type=jax.ShapeDtypeStruct((_N, 16), jnp.int32),
        mesh=mesh,
        compiler_params=cp,
    )
    def sck(d_hbm, o_hbm):
        def body(d_vmem, o_vmem):
            lane = lax.iota(jnp.int32, 16)
            inf16 = jnp.full((16,), jnp.inf, jnp.float32)
            zero16 = jnp.zeros((16,), jnp.int32)

            @pl.loop(0, _RB)
            def _row(r):
                def grp(gidx, carry):
                    base = gidx * (_G * 16)
                    vs = [d_vmem[r, pl.ds(base + 16 * i, 16)]
                          for i in range(_G)]
                    m = vs[0]
                    for i in range(1, _G):
                        m = jnp.minimum(m, vs[i])
                    mn = jnp.min(m)

                    def do_merge(c3):
                        bk, bv, bmax = c3
                        for i in range(_G):
                            ci = vs[i]

                            def m1(c2, ci=ci, i=i):
                                bk2, bv2, _ = c2
                                pi = lane + (base + 16 * i)
                                ks, ps = plsc.sort_key_val(ci, pi)
                                kr = lax.rev(ks, (0,))
                                pr = lax.rev(ps, (0,))
                                keep = bk2 <= kr
                                nk = jnp.where(keep, bk2, kr)
                                nv = jnp.where(keep, bv2, pr)
                                nk, nv = plsc.sort_key_val(nk, nv)
                                return nk, nv, jnp.max(nk)

                            bk, bv, bmax = lax.cond(
                                jnp.min(ci) < bmax, m1, lambda c2: c2,
                                (bk, bv, bmax))
                        return bk, bv, bmax

                    return lax.cond(mn < carry[2], do_merge,
                                    lambda c3: c3, carry)

                bk, bv, bmax = lax.fori_loop(
                    0, _NG, grp,
                    (inf16, zero16, jnp.float32(jnp.inf)))
                o_vmem[r, :] = bv

        pltpu.emit_pipeline(
            body,
            grid=(_N // _RB,),
            in_specs=[pl.BlockSpec((_RB, _N), lambda i: (i, 0))],
            out_specs=[pl.BlockSpec((_RB, 16), lambda i: (i, 0))],
            core_axis_name=("c", "s"),
            dimension_semantics=(pltpu.PARALLEL,),
        )(d_hbm, o_hbm)

    return sck(d)


@jax.jit
def kernel(x):
    xs = jnp.squeeze(x, -1)
    neigh = []
    for b in range(_B):
        d_b = _tc_distance(xs[b])
        idx_b = _sc_topk(d_b)
        neigh.append(idx_b[:, :_K])
    nb = jnp.stack(neigh, 0)
    centers = jnp.broadcast_to(
        jnp.arange(_N, dtype=jnp.int32)[None, :, None], (_B, _N, _K))
    return jnp.stack([nb, centers], 0)


# speedup vs baseline: 6.0917x; 6.0917x over previous
import dataclasses
import functools

import jax
import jax.numpy as jnp
from jax import lax
from jax.experimental import pallas as pl
from jax.experimental.pallas import tpu as pltpu
from jax.experimental.pallas import tpu_sc as plsc

_K = 15
_N = 4096
_C = 64
_B = 8
_R = 512
_RB = 8
_NCH = 32
_GV = 16
_NGRP = _N // (16 * _GV)
_CAP = 128


def _dist_kernel(xb_ref, xa_ref, d_ref, cm_ref):
    xb = xb_ref[...]
    xa = xa_ref[...]
    sq_all = jnp.sum(xa * xa, axis=0)
    sq_rows = jnp.sum(xb * xb, axis=0)
    g = lax.dot_general(
        xb, xa, (((0,), (0,)), ((), ())),
        preferred_element_type=jnp.float32,
        precision=lax.Precision.DEFAULT,
    )
    d = (sq_rows[:, None] + sq_all[None, :]) - 2.0 * g
    base = pl.program_id(0) * _R
    col = lax.broadcasted_iota(jnp.int32, (_R, _N), 1)
    row_g = lax.broadcasted_iota(jnp.int32, (_R, _N), 0) + base
    dm = jnp.where(col == row_g, jnp.inf, d)
    d_ref[...] = dm
    cm_ref[...] = jnp.min(dm.reshape(_R, _NCH, 128), axis=2)


def _tc_distance(xb):
    return pl.pallas_call(
        _dist_kernel,
        grid=(_N // _R,),
        in_specs=[
            pl.BlockSpec((_C, _R), lambda r: (0, r)),
            pl.BlockSpec((_C, _N), lambda r: (0, 0)),
        ],
        out_specs=[
            pl.BlockSpec((_R, _N), lambda r: (r, 0)),
            pl.BlockSpec((_R, _NCH), lambda r: (r, 0)),
        ],
        out_shape=[
            jax.ShapeDtypeStruct((_N, _N), jnp.float32),
            jax.ShapeDtypeStruct((_N, _NCH), jnp.float32),
        ],
    )(xb, xb)


def _sc_topk(d, cm):
    mesh = plsc.VectorSubcoreMesh(core_axis_name="c", subcore_axis_name="s")
    cp = pltpu.CompilerParams()
    if "needs_layout_passes" in pltpu.CompilerParams.__dataclass_fields__:
        cp = dataclasses.replace(cp, needs_layout_passes=False)

    @functools.partial(
        pl.kernel,
        out_type=jax.ShapeDtypeStruct((_N, 16), jnp.int32),
        mesh=mesh,
        compiler_params=cp,
        scratch_types=[
            pltpu.VMEM((_CAP,), jnp.float32),
            pltpu.VMEM((_CAP,), jnp.int32),
        ],
    )
    def sck(d_hbm, cm_hbm, o_hbm, cv_ref, ci_ref):
        def body(d_vmem, cm_vmem, o_vmem):
            lane = lax.iota(jnp.int32, 16)
            inf16 = jnp.full((16,), jnp.inf, jnp.float32)
            zero16 = jnp.zeros((16,), jnp.int32)

            @pl.loop(0, _RB)
            def _row(r):
                c0 = cm_vmem[r, pl.ds(0, 16)]
                c1 = cm_vmem[r, pl.ds(16, 16)]
                s0 = lax.sort(c0, dimension=0)
                s1 = lax.sort(c1, dimension=0)
                st = lax.sort(jnp.minimum(s0, lax.rev(s1, (0,))),
                              dimension=0)
                t = jnp.max(jnp.where(lane <= 14, st, -jnp.inf))
                tv = jnp.broadcast_to(t, (16,))

                for s in range(_CAP // 16):
                    cv_ref[pl.ds(16 * s, 16)] = inf16

                def grp(g, ptr):
                    base = g * (16 * _GV)
                    vls = [d_vmem[r, pl.ds(base + 16 * i, 16)]
                           for i in range(_GV)]
                    m = vls[0]
                    for i in range(1, _GV):
                        m = jnp.minimum(m, vls[i])
                    mn = jnp.min(m)

                    def compact(p):
                        msks = [vls[i] <= tv for i in range(_GV)]
                        cnts = [jnp.sum(msks[i].astype(jnp.int32))
                                for i in range(_GV)]
                        for i in range(_GV):
                            p = jnp.minimum(p, _CAP - 16)
                            plsc.store_compressed(
                                cv_ref.at[pl.ds(p, 16)], vls[i],
                                mask=msks[i])
                            plsc.store_compressed(
                                ci_ref.at[pl.ds(p, 16)],
                                lane + (base + 16 * i), mask=msks[i])
                            p = p + cnts[i]
                        return p

                    return lax.cond(mn <= t, compact, lambda p: p, ptr)

                ptr = lax.fori_loop(0, _NGRP, grp, 0)
                nvr = (jnp.minimum(ptr, _CAP) + 15) // 16

                def merge(j, c2):
                    bk, bv = c2
                    ck = cv_ref[pl.ds(16 * j, 16)]
                    cc = ci_ref[pl.ds(16 * j, 16)]
                    ks, ps = plsc.sort_key_val(ck, cc)
                    kr = lax.rev(ks, (0,))
                    pr = lax.rev(ps, (0,))
                    keep = bk <= kr
                    nk = jnp.where(keep, bk, kr)
                    nv = jnp.where(keep, bv, pr)
                    nk, nv = plsc.sort_key_val(nk, nv)
                    return (nk, nv)

                bk, bv = lax.fori_loop(0, nvr, merge, (inf16, zero16))
                o_vmem[r, :] = bv

        pltpu.emit_pipeline(
            body,
            grid=(_N // _RB,),
            in_specs=[pl.BlockSpec((_RB, _N), lambda i: (i, 0)),
                      pl.BlockSpec((_RB, _NCH), lambda i: (i, 0))],
            out_specs=[pl.BlockSpec((_RB, 16), lambda i: (i, 0))],
            core_axis_name=("c", "s"),
            dimension_semantics=(pltpu.PARALLEL,),
        )(d_hbm, cm_hbm, o_hbm)

    return sck(d, cm)


@jax.jit
def kernel(x):
    xs = jnp.squeeze(x, -1)
    neigh = []
    for b in range(_B):
        d_b, cm_b = _tc_distance(xs[b])
        idx_b = _sc_topk(d_b, cm_b)
        neigh.append(idx_b[:, :_K])
    nb = jnp.stack(neigh, 0)
    centers = jnp.broadcast_to(
        jnp.arange(_N, dtype=jnp.int32)[None, :, None], (_B, _N, _K))
    return jnp.stack([nb, centers], 0)


# speedup vs baseline: 6.2789x; 1.0307x over previous
import dataclasses
import functools

import jax
import jax.numpy as jnp
from jax import lax
from jax.experimental import pallas as pl
from jax.experimental.pallas import tpu as pltpu
from jax.experimental.pallas import tpu_sc as plsc

_K = 15
_N = 4096
_C = 64
_B = 8
_R = 512
_RB = 8
_NCH = 32
_GV = 16
_NGRP = _N // (16 * _GV)
_CAP = 128


def _dist_kernel(xb_ref, xa_ref, d_ref, cm_ref):
    xb = xb_ref[...]
    xa = xa_ref[...]
    sq_all = jnp.sum(xa * xa, axis=0)
    sq_rows = jnp.sum(xb * xb, axis=0)
    g = lax.dot_general(
        xb, xa, (((0,), (0,)), ((), ())),
        preferred_element_type=jnp.float32,
        precision=lax.Precision.DEFAULT,
    )
    d = (sq_rows[:, None] + sq_all[None, :]) - 2.0 * g
    base = pl.program_id(0) * _R
    col = lax.broadcasted_iota(jnp.int32, (_R, _N), 1)
    row_g = lax.broadcasted_iota(jnp.int32, (_R, _N), 0) + base
    dm = jnp.where(col == row_g, jnp.inf, d)
    d_ref[...] = dm
    cm_ref[...] = jnp.min(dm.reshape(_R, _NCH, 128), axis=2)


def _tc_distance(xb):
    return pl.pallas_call(
        _dist_kernel,
        grid=(_N // _R,),
        in_specs=[
            pl.BlockSpec((_C, _R), lambda r: (0, r)),
            pl.BlockSpec((_C, _N), lambda r: (0, 0)),
        ],
        out_specs=[
            pl.BlockSpec((_R, _N), lambda r: (r, 0)),
            pl.BlockSpec((_R, _NCH), lambda r: (r, 0)),
        ],
        out_shape=[
            jax.ShapeDtypeStruct((_N, _N), jnp.float32),
            jax.ShapeDtypeStruct((_N, _NCH), jnp.float32),
        ],
    )(xb, xb)


def _sc_topk(d, cm):
    mesh = plsc.VectorSubcoreMesh(core_axis_name="c", subcore_axis_name="s")
    cp = pltpu.CompilerParams()
    if "needs_layout_passes" in pltpu.CompilerParams.__dataclass_fields__:
        cp = dataclasses.replace(cp, needs_layout_passes=False)

    @functools.partial(
        pl.kernel,
        out_type=jax.ShapeDtypeStruct((_N, 16), jnp.int32),
        mesh=mesh,
        compiler_params=cp,
        scratch_types=[
            pltpu.VMEM((_CAP,), jnp.float32),
            pltpu.VMEM((_CAP,), jnp.int32),
        ],
    )
    def sck(d_hbm, cm_hbm, o_hbm, cv_ref, ci_ref):
        def body(d_vmem, cm_vmem, o_vmem):
            lane = lax.iota(jnp.int32, 16)
            inf16 = jnp.full((16,), jnp.inf, jnp.float32)
            zero16 = jnp.zeros((16,), jnp.int32)

            @pl.loop(0, _RB)
            def _row(r):
                c0 = cm_vmem[r, pl.ds(0, 16)]
                c1 = cm_vmem[r, pl.ds(16, 16)]
                s0 = lax.sort(c0, dimension=0)
                s1 = lax.sort(c1, dimension=0)
                st = lax.sort(jnp.minimum(s0, lax.rev(s1, (0,))),
                              dimension=0)
                t = jnp.max(jnp.where(lane <= 14, st, -jnp.inf))
                tv = jnp.broadcast_to(t, (16,))

                for s in range(_CAP // 16):
                    cv_ref[pl.ds(16 * s, 16)] = inf16

                def grp(g, ptr):
                    base = g * (16 * _GV)
                    vls = [d_vmem[r, pl.ds(base + 16 * i, 16)]
                           for i in range(_GV)]
                    m = vls[0]
                    for i in range(1, _GV):
                        m = jnp.minimum(m, vls[i])
                    mn = jnp.min(m)

                    def compact(p):
                        msks = [vls[i] <= tv for i in range(_GV)]
                        cnts = [plsc.all_reduce_population_count(msks[i])[0]
                                for i in range(_GV)]
                        for i in range(_GV):
                            p = jnp.minimum(p, _CAP - 16)
                            plsc.store_compressed(
                                cv_ref.at[pl.ds(p, 16)], vls[i],
                                mask=msks[i])
                            plsc.store_compressed(
                                ci_ref.at[pl.ds(p, 16)],
                                lane + (base + 16 * i), mask=msks[i])
                            p = p + cnts[i]
                        return p

                    return lax.cond(mn <= t, compact, lambda p: p, ptr)

                ptr = lax.fori_loop(0, _NGRP, grp, 0)
                nvr = (jnp.minimum(ptr, _CAP) + 15) // 16

                def merge(j, c2):
                    bk, bv = c2
                    ck = cv_ref[pl.ds(16 * j, 16)]
                    cc = ci_ref[pl.ds(16 * j, 16)]
                    ks, ps = plsc.sort_key_val(ck, cc)
                    kr = lax.rev(ks, (0,))
                    pr = lax.rev(ps, (0,))
                    keep = bk <= kr
                    nk = jnp.where(keep, bk, kr)
                    nv = jnp.where(keep, bv, pr)
                    nk, nv = plsc.sort_key_val(nk, nv)
                    return (nk, nv)

                bk, bv = lax.fori_loop(0, nvr, merge, (inf16, zero16))
                o_vmem[r, :] = bv

        pltpu.emit_pipeline(
            body,
            grid=(_N // _RB,),
            in_specs=[pl.BlockSpec((_RB, _N), lambda i: (i, 0)),
                      pl.BlockSpec((_RB, _NCH), lambda i: (i, 0))],
            out_specs=[pl.BlockSpec((_RB, 16), lambda i: (i, 0))],
            core_axis_name=("c", "s"),
            dimension_semantics=(pltpu.PARALLEL,),
        )(d_hbm, cm_hbm, o_hbm)

    return sck(d, cm)


@jax.jit
def kernel(x):
    xs = jnp.squeeze(x, -1)
    neigh = []
    for b in range(_B):
        d_b, cm_b = _tc_distance(xs[b])
        idx_b = _sc_topk(d_b, cm_b)
        neigh.append(idx_b[:, :_K])
    nb = jnp.stack(neigh, 0)
    centers = jnp.broadcast_to(
        jnp.arange(_N, dtype=jnp.int32)[None, :, None], (_B, _N, _K))
    return jnp.stack([nb, centers], 0)


# speedup vs baseline: 8.1270x; 1.2943x over previous
import dataclasses
import functools

import jax
import jax.numpy as jnp
from jax import lax
from jax.experimental import pallas as pl
from jax.experimental.pallas import tpu as pltpu
from jax.experimental.pallas import tpu_sc as plsc

_K = 15
_N = 4096
_C = 64
_B = 8
_TCB = 2
_R = 512
_RB = 8
_NCH = 32
_GV = 16
_NGRP = _N // (16 * _GV)
_CAP = 128


def _fused_kernel(xb_ref, xa_ref, out_ref, d_ref):
    xb = xb_ref[...]
    xa = xa_ref[...]
    sq_all = jnp.sum(xa * xa, axis=0)
    sq_rows = jnp.sum(xb * xb, axis=0)
    g = lax.dot_general(
        xb, xa, (((0,), (0,)), ((), ())),
        preferred_element_type=jnp.float32,
        precision=lax.Precision.DEFAULT,
    )
    d = (sq_rows[:, None] + sq_all[None, :]) - 2.0 * g
    base = pl.program_id(0) * _R
    col = lax.broadcasted_iota(jnp.int32, (_R, _N), 1)
    row_g = lax.broadcasted_iota(jnp.int32, (_R, _N), 0) + base
    d_ref[...] = jnp.where(col == row_g, jnp.inf, d)

    kcol = lax.broadcasted_iota(jnp.int32, (_R, _K), 1)

    def extract(k, acc):
        dk = d_ref[...]
        m = jnp.min(dk, axis=1)
        eq = dk == m[:, None]
        idx = jnp.min(jnp.where(eq, col, _N), axis=1)
        acc = jnp.where(kcol == k, idx[:, None], acc)
        d_ref[...] = jnp.where(col == idx[:, None], jnp.inf, dk)
        return acc

    out_ref[...] = lax.fori_loop(
        0, _K, extract, jnp.zeros((_R, _K), jnp.int32))


def _tc_fused(xb):
    return pl.pallas_call(
        _fused_kernel,
        grid=(_N // _R,),
        in_specs=[
            pl.BlockSpec((_C, _R), lambda r: (0, r)),
            pl.BlockSpec((_C, _N), lambda r: (0, 0)),
        ],
        out_specs=pl.BlockSpec((_R, _K), lambda r: (r, 0)),
        out_shape=jax.ShapeDtypeStruct((_N, _K), jnp.int32),
        scratch_shapes=[pltpu.VMEM((_R, _N), jnp.float32)],
    )(xb, xb)


def _dist_kernel(xb_ref, xa_ref, d_ref, cm_ref):
    xb = xb_ref[...]
    xa = xa_ref[...]
    sq_all = jnp.sum(xa * xa, axis=0)
    sq_rows = jnp.sum(xb * xb, axis=0)
    g = lax.dot_general(
        xb, xa, (((0,), (0,)), ((), ())),
        preferred_element_type=jnp.float32,
        precision=lax.Precision.DEFAULT,
    )
    d = (sq_rows[:, None] + sq_all[None, :]) - 2.0 * g
    base = pl.program_id(0) * _R
    col = lax.broadcasted_iota(jnp.int32, (_R, _N), 1)
    row_g = lax.broadcasted_iota(jnp.int32, (_R, _N), 0) + base
    dm = jnp.where(col == row_g, jnp.inf, d)
    d_ref[...] = dm
    cm_ref[...] = jnp.min(dm.reshape(_R, _NCH, 128), axis=2)


def _tc_distance(xb):
    return pl.pallas_call(
        _dist_kernel,
        grid=(_N // _R,),
        in_specs=[
            pl.BlockSpec((_C, _R), lambda r: (0, r)),
            pl.BlockSpec((_C, _N), lambda r: (0, 0)),
        ],
        out_specs=[
            pl.BlockSpec((_R, _N), lambda r: (r, 0)),
            pl.BlockSpec((_R, _NCH), lambda r: (r, 0)),
        ],
        out_shape=[
            jax.ShapeDtypeStruct((_N, _N), jnp.float32),
            jax.ShapeDtypeStruct((_N, _NCH), jnp.float32),
        ],
    )(xb, xb)


def _sc_topk(d, cm):
    mesh = plsc.VectorSubcoreMesh(core_axis_name="c", subcore_axis_name="s")
    cp = pltpu.CompilerParams()
    if "needs_layout_passes" in pltpu.CompilerParams.__dataclass_fields__:
        cp = dataclasses.replace(cp, needs_layout_passes=False)

    @functools.partial(
        pl.kernel,
        out_type=jax.ShapeDtypeStruct((_N, 16), jnp.int32),
        mesh=mesh,
        compiler_params=cp,
        scratch_types=[
            pltpu.VMEM((_CAP,), jnp.float32),
            pltpu.VMEM((_CAP,), jnp.int32),
        ],
    )
    def sck(d_hbm, cm_hbm, o_hbm, cv_ref, ci_ref):
        def body(d_vmem, cm_vmem, o_vmem):
            lane = lax.iota(jnp.int32, 16)
            inf16 = jnp.full((16,), jnp.inf, jnp.float32)
            zero16 = jnp.zeros((16,), jnp.int32)

            @pl.loop(0, _RB)
            def _row(r):
                c0 = cm_vmem[r, pl.ds(0, 16)]
                c1 = cm_vmem[r, pl.ds(16, 16)]
                s0 = lax.sort(c0, dimension=0)
                s1 = lax.sort(c1, dimension=0)
                st = lax.sort(jnp.minimum(s0, lax.rev(s1, (0,))),
                              dimension=0)
                t = jnp.max(jnp.where(lane <= 14, st, -jnp.inf))
                tv = jnp.broadcast_to(t, (16,))

                for s in range(_CAP // 16):
                    cv_ref[pl.ds(16 * s, 16)] = inf16

                def grp(g, ptr):
                    base = g * (16 * _GV)
                    vls = [d_vmem[r, pl.ds(base + 16 * i, 16)]
                           for i in range(_GV)]
                    m = vls[0]
                    for i in range(1, _GV):
                        m = jnp.minimum(m, vls[i])
                    mn = jnp.min(m)

                    def compact(p):
                        msks = [vls[i] <= tv for i in range(_GV)]
                        cnts = [plsc.all_reduce_population_count(msks[i])[0]
                                for i in range(_GV)]
                        for i in range(_GV):
                            p = jnp.minimum(p, _CAP - 16)
                            plsc.store_compressed(
                                cv_ref.at[pl.ds(p, 16)], vls[i],
                                mask=msks[i])
                            plsc.store_compressed(
                                ci_ref.at[pl.ds(p, 16)],
                                lane + (base + 16 * i), mask=msks[i])
                            p = p + cnts[i]
                        return p

                    return lax.cond(mn <= t, compact, lambda p: p, ptr)

                ptr = lax.fori_loop(0, _NGRP, grp, 0)
                nvr = (jnp.minimum(ptr, _CAP) + 15) // 16

                def merge(j, c2):
                    bk, bv = c2
                    ck = cv_ref[pl.ds(16 * j, 16)]
                    cc = ci_ref[pl.ds(16 * j, 16)]
                    ks, ps = plsc.sort_key_val(ck, cc)
                    kr = lax.rev(ks, (0,))
                    pr = lax.rev(ps, (0,))
                    keep = bk <= kr
                    nk = jnp.where(keep, bk, kr)
                    nv = jnp.where(keep, bv, pr)
                    nk, nv = plsc.sort_key_val(nk, nv)
                    return (nk, nv)

                bk, bv = lax.fori_loop(0, nvr, merge, (inf16, zero16))
                o_vmem[r, :] = bv

        pltpu.emit_pipeline(
            body,
            grid=(_N // _RB,),
            in_specs=[pl.BlockSpec((_RB, _N), lambda i: (i, 0)),
                      pl.BlockSpec((_RB, _NCH), lambda i: (i, 0))],
            out_specs=[pl.BlockSpec((_RB, 16), lambda i: (i, 0))],
            core_axis_name=("c", "s"),
            dimension_semantics=(pltpu.PARALLEL,),
        )(d_hbm, cm_hbm, o_hbm)

    return sck(d, cm)


@jax.jit
def kernel(x):
    xs = jnp.squeeze(x, -1)
    neigh = [None] * _B
    for b in range(_TCB, _B):
        d_b, cm_b = _tc_distance(xs[b])
        neigh[b] = _sc_topk(d_b, cm_b)[:, :_K]
    for b in range(_TCB):
        neigh[b] = _tc_fused(xs[b])
    nb = jnp.stack(neigh, 0)
    centers = jnp.broadcast_to(
        jnp.arange(_N, dtype=jnp.int32)[None, :, None], (_B, _N, _K))
    return jnp.stack([nb, centers], 0)


# speedup vs baseline: 8.2801x; 1.0188x over previous
import dataclasses
import functools

import jax
import jax.numpy as jnp
from jax import lax
from jax.experimental import pallas as pl
from jax.experimental.pallas import tpu as pltpu
from jax.experimental.pallas import tpu_sc as plsc

_K = 15
_N = 4096
_C = 64
_B = 8
_TCB = 2
_R = 512
_RB = 8
_NCH = 32
_GV = 16
_NGRP = _N // (16 * _GV)
_CAP = 128


def _fused_kernel(xb_ref, xa_ref, out_ref, d_ref):
    xb = xb_ref[...]
    xa = xa_ref[...]
    sq_all = jnp.sum(xa * xa, axis=0)
    sq_rows = jnp.sum(xb * xb, axis=0)
    g = lax.dot_general(
        xb, xa, (((0,), (0,)), ((), ())),
        preferred_element_type=jnp.float32,
        precision=lax.Precision.DEFAULT,
    )
    d = (sq_rows[:, None] + sq_all[None, :]) - 2.0 * g
    base = pl.program_id(0) * _R
    col = lax.broadcasted_iota(jnp.int32, (_R, _N), 1)
    row_g = lax.broadcasted_iota(jnp.int32, (_R, _N), 0) + base
    dm = jnp.where(col == row_g, jnp.inf, d)
    d_ref[...] = dm

    kcol = lax.broadcasted_iota(jnp.int32, (_R, _K), 1)

    def extract(k, carry):
        m, acc = carry
        dk = d_ref[...]
        eq = dk == m[:, None]
        idx = jnp.min(jnp.where(eq, col, _N), axis=1)
        acc = jnp.where(kcol == k, idx[:, None], acc)
        nd = jnp.where(col == idx[:, None], jnp.inf, dk)
        d_ref[...] = nd
        return (jnp.min(nd, axis=1), acc)

    _, out = lax.fori_loop(
        0, _K, extract,
        (jnp.min(dm, axis=1), jnp.zeros((_R, _K), jnp.int32)))
    out_ref[...] = out


def _tc_fused(xb):
    return pl.pallas_call(
        _fused_kernel,
        grid=(_N // _R,),
        in_specs=[
            pl.BlockSpec((_C, _R), lambda r: (0, r)),
            pl.BlockSpec((_C, _N), lambda r: (0, 0)),
        ],
        out_specs=pl.BlockSpec((_R, _K), lambda r: (r, 0)),
        out_shape=jax.ShapeDtypeStruct((_N, _K), jnp.int32),
        scratch_shapes=[pltpu.VMEM((_R, _N), jnp.float32)],
    )(xb, xb)


def _dist_kernel(xb_ref, xa_ref, d_ref, cm_ref):
    xb = xb_ref[...]
    xa = xa_ref[...]
    sq_all = jnp.sum(xa * xa, axis=0)
    sq_rows = jnp.sum(xb * xb, axis=0)
    g = lax.dot_general(
        xb, xa, (((0,), (0,)), ((), ())),
        preferred_element_type=jnp.float32,
        precision=lax.Precision.DEFAULT,
    )
    d = (sq_rows[:, None] + sq_all[None, :]) - 2.0 * g
    base = pl.program_id(0) * _R
    col = lax.broadcasted_iota(jnp.int32, (_R, _N), 1)
    row_g = lax.broadcasted_iota(jnp.int32, (_R, _N), 0) + base
    dm = jnp.where(col == row_g, jnp.inf, d)
    d_ref[...] = dm
    cm_ref[...] = jnp.min(dm.reshape(_R, _NCH, 128), axis=2)


def _tc_distance(xb):
    return pl.pallas_call(
        _dist_kernel,
        grid=(_N // _R,),
        in_specs=[
            pl.BlockSpec((_C, _R), lambda r: (0, r)),
            pl.BlockSpec((_C, _N), lambda r: (0, 0)),
        ],
        out_specs=[
            pl.BlockSpec((_R, _N), lambda r: (r, 0)),
            pl.BlockSpec((_R, _NCH), lambda r: (r, 0)),
        ],
        out_shape=[
            jax.ShapeDtypeStruct((_N, _N), jnp.float32),
            jax.ShapeDtypeStruct((_N, _NCH), jnp.float32),
        ],
    )(xb, xb)


def _sc_topk(d, cm):
    mesh = plsc.VectorSubcoreMesh(core_axis_name="c", subcore_axis_name="s")
    cp = pltpu.CompilerParams()
    if "needs_layout_passes" in pltpu.CompilerParams.__dataclass_fields__:
        cp = dataclasses.replace(cp, needs_layout_passes=False)

    @functools.partial(
        pl.kernel,
        out_type=jax.ShapeDtypeStruct((_N, 16), jnp.int32),
        mesh=mesh,
        compiler_params=cp,
        scratch_types=[
            pltpu.VMEM((_CAP,), jnp.float32),
            pltpu.VMEM((_CAP,), jnp.int32),
        ],
    )
    def sck(d_hbm, cm_hbm, o_hbm, cv_ref, ci_ref):
        def body(d_vmem, cm_vmem, o_vmem):
            lane = lax.iota(jnp.int32, 16)
            inf16 = jnp.full((16,), jnp.inf, jnp.float32)
            zero16 = jnp.zeros((16,), jnp.int32)

            @pl.loop(0, _RB)
            def _row(r):
                c0 = cm_vmem[r, pl.ds(0, 16)]
                c1 = cm_vmem[r, pl.ds(16, 16)]
                s0 = lax.sort(c0, dimension=0)
                s1 = lax.sort(c1, dimension=0)
                st = lax.sort(jnp.minimum(s0, lax.rev(s1, (0,))),
                              dimension=0)
                t = jnp.max(jnp.where(lane <= 14, st, -jnp.inf))
                tv = jnp.broadcast_to(t, (16,))

                for s in range(_CAP // 16):
                    cv_ref[pl.ds(16 * s, 16)] = inf16

                def grp(g, ptr):
                    base = g * (16 * _GV)
                    vls = [d_vmem[r, pl.ds(base + 16 * i, 16)]
                           for i in range(_GV)]
                    m = vls[0]
                    for i in range(1, _GV):
                        m = jnp.minimum(m, vls[i])
                    hit = plsc.all_reduce_population_count(m <= tv)[0]

                    def compact(p):
                        msks = [vls[i] <= tv for i in range(_GV)]
                        cnts = [plsc.all_reduce_population_count(msks[i])[0]
                                for i in range(_GV)]
                        for i in range(_GV):
                            p = jnp.minimum(p, _CAP - 16)
                            plsc.store_compressed(
                                cv_ref.at[pl.ds(p, 16)], vls[i],
                                mask=msks[i])
                            plsc.store_compressed(
                                ci_ref.at[pl.ds(p, 16)],
                                lane + (base + 16 * i), mask=msks[i])
                            p = p + cnts[i]
                        return p

                    return lax.cond(hit > 0, compact, lambda p: p, ptr)

                ptr = lax.fori_loop(0, _NGRP, grp, 0)
                nvr = (jnp.minimum(ptr, _CAP) + 15) // 16

                def merge(j, c2):
                    bk, bv = c2
                    ck = cv_ref[pl.ds(16 * j, 16)]
                    cc = ci_ref[pl.ds(16 * j, 16)]
                    ks, ps = plsc.sort_key_val(ck, cc)
                    kr = lax.rev(ks, (0,))
                    pr = lax.rev(ps, (0,))
                    keep = bk <= kr
                    nk = jnp.where(keep, bk, kr)
                    nv = jnp.where(keep, bv, pr)
                    nk, nv = plsc.sort_key_val(nk, nv)
                    return (nk, nv)

                bk, bv = lax.fori_loop(0, nvr, merge, (inf16, zero16))
                o_vmem[r, :] = bv

        pltpu.emit_pipeline(
            body,
            grid=(_N // _RB,),
            in_specs=[pl.BlockSpec((_RB, _N), lambda i: (i, 0)),
                      pl.BlockSpec((_RB, _NCH), lambda i: (i, 0))],
            out_specs=[pl.BlockSpec((_RB, 16), lambda i: (i, 0))],
            core_axis_name=("c", "s"),
            dimension_semantics=(pltpu.PARALLEL,),
        )(d_hbm, cm_hbm, o_hbm)

    return sck(d, cm)


@jax.jit
def kernel(x):
    xs = jnp.squeeze(x, -1)
    neigh = [None] * _B
    for b in range(_TCB, _B):
        d_b, cm_b = _tc_distance(xs[b])
        neigh[b] = _sc_topk(d_b, cm_b)[:, :_K]
    for b in range(_TCB):
        neigh[b] = _tc_fused(xs[b])
    nb = jnp.stack(neigh, 0)
    centers = jnp.broadcast_to(
        jnp.arange(_N, dtype=jnp.int32)[None, :, None], (_B, _N, _K))
    return jnp.stack([nb, centers], 0)
